# baseline (speedup 1.0000x reference)
import functools

import jax
import jax.numpy as jnp
from jax import lax
from jax.experimental import pallas as pl
from jax.experimental.pallas import tpu as pltpu
from jax.experimental.pallas import tpu_sc as plsc

NUM_CLASSES = 100000
FEAT_DIM = 64
BATCH = 16384

NC = 2
NS = 16
L = 16
NW = NC * NS
BPW = BATCH // NW
CH = 128
NCHUNK = BPW // CH
COLV = FEAT_DIM // L


def _body(feat_hbm, lab_hbm, cent_hbm, out_hbm, idx_v, feat_v, rows_v,
          acc_v, sem):
    c = lax.axis_index("c")
    s = lax.axis_index("s")
    wid = s * NC + c
    base = wid * BPW

    for j in range(NCHUNK):
        pltpu.sync_copy(lab_hbm.at[pl.ds(base + j * CH, CH)], idx_v.at[j])

    descs = [
        pltpu.async_copy(cent_hbm.at[idx_v.at[j]],
                         rows_v.at[pl.ds(j * CH, CH)], sem)
        for j in range(NCHUNK)
    ]
    pltpu.sync_copy(feat_hbm.at[pl.ds(base, BPW)], feat_v)
    for d in descs:
        d.wait()

    zero = jnp.zeros((L,), jnp.float32)

    def row(i, accs):
        out = []
        for k in range(COLV):
            f = feat_v[i, pl.ds(k * L, L)]
            r = rows_v[i, pl.ds(k * L, L)]
            d = f - r
            out.append(accs[k] + d * d)
        return tuple(out)

    accs = lax.fori_loop(0, BPW, row, (zero,) * COLV)
    acc_v[...] = (accs[0] + accs[1]) + (accs[2] + accs[3])
    pltpu.sync_copy(acc_v, out_hbm.at[wid])


_partials = functools.partial(
    pl.kernel,
    out_type=jax.ShapeDtypeStruct((NW, L), jnp.float32),
    mesh=plsc.VectorSubcoreMesh(core_axis_name="c", subcore_axis_name="s",
                                num_cores=NC, num_subcores=NS),
    scratch_types=[
        pltpu.VMEM((NCHUNK, CH), jnp.int32),
        pltpu.VMEM((BPW, FEAT_DIM), jnp.float32),
        pltpu.VMEM((BPW, FEAT_DIM), jnp.float32),
        pltpu.VMEM((L,), jnp.float32),
        pltpu.SemaphoreType.DMA,
    ],
    compiler_params=pltpu.CompilerParams(use_tc_tiling_on_sc=False),
)(_body)


@jax.jit
def kernel(features, labels, centers):
    batch_size = features.shape[0]
    partials = _partials(features, labels.astype(jnp.int32), centers)
    return jnp.sum(partials) / (2.0 * batch_size)


# speedup vs baseline: 1.3468x; 1.3468x over previous
import functools

import jax
import jax.numpy as jnp
from jax import lax
from jax.experimental import pallas as pl
from jax.experimental.pallas import tpu as pltpu
from jax.experimental.pallas import tpu_sc as plsc

NUM_CLASSES = 100000
FEAT_DIM = 64
BATCH = 16384

NC = 2
NS = 16
L = 16
NW = NC * NS
BPW = BATCH // NW
CHUNK = 256
NPHASE = BPW // CHUNK
COLV = FEAT_DIM // L


def _body(feat_hbm, lab_hbm, cent_hbm, out_hbm, idx_v, feat_v, rows_v,
          acc_v, fsem, rsem):
    c = lax.axis_index("c")
    s = lax.axis_index("s")
    wid = s * NC + c
    base = wid * BPW

    pltpu.sync_copy(lab_hbm.at[pl.ds(base, BPW)], idx_v)

    def enq(g, _):
        labv = idx_v[pl.ds(g * L, L)]
        for j in range(L):
            lab = labv[j]
            pltpu.async_copy(cent_hbm.at[pl.ds(lab, 1)],
                             rows_v.at[pl.ds(g * L + j, 1)], rsem)
        return 0

    lax.fori_loop(0, BPW // L, enq, 0)

    zero = jnp.zeros((L,), jnp.float32)
    accs = (zero,) * COLV

    for phase in range(NPHASE):
        fd = pltpu.async_copy(feat_hbm.at[pl.ds(base + phase * CHUNK, CHUNK)],
                              feat_v, fsem)

        def drain(i, _):
            pltpu.make_async_copy(cent_hbm.at[pl.ds(0, 1)],
                                  rows_v.at[pl.ds(0, 1)], rsem).wait()
            return 0

        lax.fori_loop(0, CHUNK, drain, 0)
        fd.wait()

        def row(i, carry):
            out = []
            for k in range(COLV):
                f = feat_v[i, pl.ds(k * L, L)]
                r = rows_v[phase * CHUNK + i, pl.ds(k * L, L)]
                d = f - r
                out.append(carry[k] + d * d)
            return tuple(out)

        accs = lax.fori_loop(0, CHUNK, row, accs)

    acc_v[...] = (accs[0] + accs[1]) + (accs[2] + accs[3])
    pltpu.sync_copy(acc_v, out_hbm.at[wid])


_partials = functools.partial(
    pl.kernel,
    out_type=jax.ShapeDtypeStruct((NW, L), jnp.float32),
    mesh=plsc.VectorSubcoreMesh(core_axis_name="c", subcore_axis_name="s",
                                num_cores=NC, num_subcores=NS),
    scratch_types=[
        pltpu.VMEM((BPW,), jnp.int32),
        pltpu.VMEM((CHUNK, FEAT_DIM), jnp.float32),
        pltpu.VMEM((BPW, FEAT_DIM), jnp.float32),
        pltpu.VMEM((L,), jnp.float32),
        pltpu.SemaphoreType.DMA,
        pltpu.SemaphoreType.DMA,
    ],
)(_body)


@jax.jit
def kernel(features, labels, centers):
    batch_size = features.shape[0]
    partials = _partials(features, labels.astype(jnp.int32), centers)
    return jnp.sum(partials) / (2.0 * batch_size)


# speedup vs baseline: 2.1758x; 1.6156x over previous
import functools

import jax
import jax.numpy as jnp
from jax import lax
from jax.experimental import pallas as pl
from jax.experimental.pallas import tpu as pltpu
from jax.experimental.pallas import tpu_sc as plsc

NUM_CLASSES = 100000
FEAT_DIM = 64
BATCH = 16384

NC = 2
NS = 16
L = 16
NW = NC * NS
DPW = FEAT_DIM // NW
BCHUNK = 4096
NBCH = BATCH // BCHUNK
UNROLL = 4
HALF_A = 50048
HALF_B = NUM_CLASSES - HALF_A


def _body(feat_hbm, lab_hbm, cent_hbm, out_hbm, ta_v, tb_v, lab_v, f_v,
          acc_v, tsema, tsemb, lsem, fsem0, fsem1):
    c = lax.axis_index("c")
    s = lax.axis_index("s")
    wid = s * NC + c
    fsems = (fsem0, fsem1)

    seq = [(d, p, ch) for d in range(DPW) for p in range(2)
           for ch in range(NBCH)]

    def issue_f(k):
        d, _, ch = seq[k]
        return pltpu.async_copy(
            feat_hbm.at[wid * DPW + d, pl.ds(ch * BCHUNK, BCHUNK)],
            f_v.at[k % 2], fsems[k % 2])

    def issue_ta(d):
        return pltpu.async_copy(cent_hbm.at[wid * DPW + d, pl.ds(0, HALF_A)],
                                ta_v, tsema)

    def issue_tb(d):
        return pltpu.async_copy(
            cent_hbm.at[wid * DPW + d, pl.ds(HALF_A, HALF_B)], tb_v, tsemb)

    tda = issue_ta(0)
    lb = pltpu.async_copy(lab_hbm, lab_v, lsem)
    fdesc = {0: issue_f(0), 1: issue_f(1)}
    tdb = issue_tb(0)
    lb.wait()

    zero = jnp.zeros((L,), jnp.float32)
    accs = (zero,) * UNROLL

    for k, (d, p, ch) in enumerate(seq):
        if p == 0 and ch == 0:
            tda.wait()
        if p == 1 and ch == 0:
            tdb.wait()
        fdesc[k].wait()
        cbase = ch * BCHUNK
        fb = k % 2

        def group(g, carry):
            out = []
            for u in range(UNROLL):
                off = (g * UNROLL + u) * L
                idx = lab_v[pl.ds(cbase + off, L)]
                fv = f_v[fb, pl.ds(off, L)]
                if p == 0:
                    m = idx < HALF_A
                    tv = plsc.load_gather(ta_v, [idx], mask=m)
                else:
                    m = idx >= HALF_A
                    tv = plsc.load_gather(tb_v, [idx - HALF_A], mask=m)
                dd = fv - tv
                out.append(carry[u] + jnp.where(m, dd * dd, 0.0))
            return tuple(out)

        accs = lax.fori_loop(0, BCHUNK // (L * UNROLL), group, accs)

        if k + 2 < len(seq):
            fdesc[k + 2] = issue_f(k + 2)
        if d == 0 and p == 0 and ch == NBCH - 1 and DPW > 1:
            tda = issue_ta(1)
        if d == 0 and p == 1 and ch == NBCH - 1 and DPW > 1:
            tdb = issue_tb(1)

    acc_v[...] = (accs[0] + accs[1]) + (accs[2] + accs[3])
    pltpu.sync_copy(acc_v, out_hbm.at[wid])


_partials = functools.partial(
    pl.kernel,
    out_type=jax.ShapeDtypeStruct((NW, L), jnp.float32),
    mesh=plsc.VectorSubcoreMesh(core_axis_name="c", subcore_axis_name="s",
                                num_cores=NC, num_subcores=NS),
    scratch_types=[
        pltpu.VMEM((HALF_A,), jnp.float32),
        pltpu.VMEM((HALF_B,), jnp.float32),
        pltpu.VMEM((BATCH,), jnp.int32),
        pltpu.VMEM((2, BCHUNK), jnp.float32),
        pltpu.VMEM((L,), jnp.float32),
        pltpu.SemaphoreType.DMA,
        pltpu.SemaphoreType.DMA,
        pltpu.SemaphoreType.DMA,
        pltpu.SemaphoreType.DMA,
        pltpu.SemaphoreType.DMA,
    ],
    compiler_params=pltpu.CompilerParams(needs_layout_passes=False),
)(_body)


@jax.jit
def kernel(features, labels, centers):
    batch_size = features.shape[0]
    partials = _partials(features.T, labels.astype(jnp.int32), centers.T)
    return jnp.sum(partials) / (2.0 * batch_size)
